# Initial kernel scaffold; baseline (speedup 1.0000x reference)
import jax
import jax.numpy as jnp
from jax.experimental import pallas as pl


def kernel(x, edge_attr, edge_weight, params, edge_index, batch):
    raise NotImplementedError("write your pallas kernel here")


# speedup vs baseline: 1.0323x; 1.0323x over previous
import jax
import jax.numpy as jnp
from jax.experimental import pallas as pl


def _dot(a, b):
    return jax.lax.dot_general(a, b, (((1,), (0,)), ((), ())),
                               precision=jax.lax.Precision.HIGHEST,
                               preferred_element_type=jnp.float32)


def _head_body(g_ref, w1, b1, w2, b2, w3, b3, w4, b4, out_ref):
    g = g_ref[...]
    g = jnp.maximum(_dot(g, w1[...]) + b1[...], 0.0)
    g = jnp.maximum(_dot(g, w2[...]) + b2[...], 0.0)
    g = jnp.maximum(_dot(g, w3[...]) + b3[...], 0.0)
    out_ref[...] = _dot(g, w4[...]) + b4[...]


def _apply_lin(p, h):
    return h @ p["w"] + p["b"]


def kernel(x, edge_attr, edge_weight, params, edge_index, batch):
    h = x
    src = edge_index[0]
    dst = edge_index[1]
    n = x.shape[0]
    for conv_p, bn_p in zip(params["convs"], params["bns"]):
        e = _apply_lin(conv_p["be2"], jax.nn.relu(_apply_lin(conv_p["be1"], edge_attr)))
        m = jax.nn.relu(h[src] + e) * edge_weight[:, None]
        agg = jax.ops.segment_sum(m, dst, num_segments=n)
        hh = (1.0 + conv_p["eps"]) * h + agg
        hh = _apply_lin(conv_p["mlp2"], jax.nn.relu(_apply_lin(conv_p["mlp1"], hh)))
        mu = jnp.mean(hh, axis=0)
        var = jnp.var(hh, axis=0)
        hh = (hh - mu) * jax.lax.rsqrt(var + 1e-5) * bn_p["gamma"] + bn_p["beta"]
        h = jax.nn.relu(hh)

    num_graphs = 256
    s = jax.ops.segment_sum(h, batch, num_segments=num_graphs)
    c = jax.ops.segment_sum(jnp.ones((n, 1), h.dtype), batch, num_segments=num_graphs)
    g = s / jnp.maximum(c, 1.0)

    p = params
    out = pl.pallas_call(
        _head_body,
        out_shape=jax.ShapeDtypeStruct((num_graphs, 1), jnp.float32),
    )(g, p["fc1"]["w"], p["fc1"]["b"], p["fc2"]["w"], p["fc2"]["b"],
      p["fc3"]["w"], p["fc3"]["b"], p["fc4"]["w"], p["fc4"]["b"])
    return out.reshape(-1)


# speedup vs baseline: 1.1490x; 1.1131x over previous
import functools

import jax
import jax.numpy as jnp
from jax import lax
from jax.experimental import pallas as pl
from jax.experimental.pallas import tpu as pltpu
from jax.experimental.pallas import tpu_sc as plsc

N = 50000
E = 800000
G = 256

_HI = jax.lax.Precision.HIGHEST


def _dot(a, b):
    return jax.lax.dot_general(a, b, (((1,), (0,)), ((), ())),
                               precision=_HI, preferred_element_type=jnp.float32)


def _dotT(a, b):
    return jax.lax.dot_general(a, b, (((0,), (0,)), ((), ())),
                               precision=_HI, preferred_element_type=jnp.float32)


@functools.lru_cache(maxsize=None)
def _make_msgpass(w):
    NT = 16
    BLK = 400
    EPT = E // NT
    NBLK = EPT // BLK
    NPT = 3136
    NP = NT * NPT
    ZR = 392
    NH = w // 16

    mesh = plsc.VectorSubcoreMesh(core_axis_name="c", subcore_axis_name="s")

    @functools.partial(
        pl.kernel, mesh=mesh,
        compiler_params=pltpu.CompilerParams(use_tc_tiling_on_sc=False),
        out_type=[jax.ShapeDtypeStruct((NP, w), jnp.float32),
                  jax.ShapeDtypeStruct((NP, w), jnp.float32)],
        scratch_types=[
            pltpu.VMEM((BLK,), jnp.int32),
            pltpu.VMEM((BLK,), jnp.int32),
            pltpu.VMEM((BLK,), jnp.float32),
            pltpu.VMEM((BLK, w), jnp.float32),
            pltpu.VMEM((BLK, w), jnp.float32),
            pltpu.VMEM_SHARED((NP, w), jnp.float32),
            pltpu.SemaphoreType.DMA,
        ],
    )
    def msgpass(hlo, hhi, elo, ehi, ei, ew, out_lo, out_hi,
                src_v, dst_v, w_v, xs_v, e_v, agg, sem):
        c = lax.axis_index("c")
        s = lax.axis_index("s")

        def zrow(i, carry):
            for hh in range(NH):
                xs_v[i, pl.ds(hh * 16, 16)] = jnp.zeros((16,), jnp.float32)
            return carry
        lax.fori_loop(0, ZR, zrow, 0)
        for r in range(NPT // ZR):
            pltpu.sync_copy(xs_v.at[pl.ds(0, ZR)],
                            agg.at[pl.ds(s * NPT + r * ZR, ZR)])
        plsc.subcore_barrier()

        base0 = s * EPT

        def blk(b, carry):
            base = base0 + b * BLK
            pltpu.sync_copy(ei.at[0, pl.ds(base, BLK)], src_v)
            pltpu.sync_copy(ei.at[1, pl.ds(base, BLK)], dst_v)
            pltpu.sync_copy(ew.at[pl.ds(base, BLK)], w_v)

            @pl.when(c == 0)
            def _():
                pltpu.sync_copy(elo.at[pl.ds(base, BLK)], e_v)
                pltpu.async_copy(hlo.at[src_v], xs_v, sem).wait()

            @pl.when(c == 1)
            def _():
                pltpu.sync_copy(ehi.at[pl.ds(base, BLK)], e_v)
                pltpu.async_copy(hhi.at[src_v], xs_v, sem).wait()

            def grp(g, carry2):
                j0 = g * 16
                wv = w_v[pl.ds(j0, 16)]
                for k in range(16):
                    wj = wv[k]
                    for hh in range(NH):
                        t = (xs_v[j0 + k, pl.ds(hh * 16, 16)]
                             + e_v[j0 + k, pl.ds(hh * 16, 16)])
                        xs_v[j0 + k, pl.ds(hh * 16, 16)] = (
                            jnp.maximum(t, 0.0) * wj)
                return carry2
            lax.fori_loop(0, BLK // 16, grp, 0)

            pltpu.sync_copy(xs_v, agg.at[dst_v], add=True)
            return carry
        lax.fori_loop(0, NBLK, blk, 0)

        plsc.subcore_barrier()

        @pl.when(c == 0)
        def _():
            pltpu.sync_copy(agg.at[pl.ds(s * NPT, NPT)],
                            out_lo.at[pl.ds(s * NPT, NPT)])

        @pl.when(c == 1)
        def _():
            pltpu.sync_copy(agg.at[pl.ds(s * NPT, NPT)],
                            out_hi.at[pl.ds(s * NPT, NPT)])

    return msgpass


_BE = 4000
_BN = 2000


def _edge_body(wp, ea_ref, w1, b1, w2, b2, olo, ohi):
    a = ea_ref[...]
    t = jnp.maximum(_dot(a, w1[...]) + b1[...], 0.0)
    ee = _dot(t, w2[...]) + b2[...]
    olo[...] = ee[:, :wp]
    ohi[...] = ee[:, wp:]


def _edge_mlp(ea, w1, b1, w2, b2, wp):
    d1p = 2 * wp
    grid = (E // _BE,)
    return pl.pallas_call(
        functools.partial(_edge_body, wp),
        grid=grid,
        in_specs=[
            pl.BlockSpec((_BE, 3), lambda i: (i, 0)),
            pl.BlockSpec((3, d1p), lambda i: (0, 0)),
            pl.BlockSpec((1, d1p), lambda i: (0, 0)),
            pl.BlockSpec((d1p, d1p), lambda i: (0, 0)),
            pl.BlockSpec((1, d1p), lambda i: (0, 0)),
        ],
        out_specs=[pl.BlockSpec((_BE, wp), lambda i: (i, 0)),
                   pl.BlockSpec((_BE, wp), lambda i: (i, 0))],
        out_shape=[jax.ShapeDtypeStruct((E, wp), jnp.float32),
                   jax.ShapeDtypeStruct((E, wp), jnp.float32)],
    )(ea, w1, b1, w2, b2)


def _nk1_body(hlo, hhi, alo, ahi, eps, m1w, m1b, m2w, m2b, ulo, uhi, stats):
    i = pl.program_id(0)
    h = jnp.concatenate([hlo[...], hhi[...]], axis=1)
    agg = jnp.concatenate([alo[...], ahi[...]], axis=1)
    z = (1.0 + eps[0, 0]) * h + agg
    t = jnp.maximum(_dot(z, m1w[...]) + m1b[...], 0.0)
    u = _dot(t, m2w[...]) + m2b[...]
    ulo[...] = u[:, :32]
    uhi[...] = u[:, 32:]

    @pl.when(i == 0)
    def _():
        stats[...] = jnp.zeros_like(stats)
    su = jnp.sum(u, axis=0, keepdims=True)
    sq = jnp.sum(u * u, axis=0, keepdims=True)
    stats[0:1, :] += su
    stats[1:2, :] += sq


def _node_mlp(hlo, hhi, alo, ahi, eps, m1w, m1b, m2w, m2b):
    wp = hlo.shape[1]
    d1p = 2 * wp
    grid = (N // _BN,)
    return pl.pallas_call(
        _nk1_body,
        grid=grid,
        in_specs=[
            pl.BlockSpec((_BN, wp), lambda i: (i, 0)),
            pl.BlockSpec((_BN, wp), lambda i: (i, 0)),
            pl.BlockSpec((_BN, wp), lambda i: (i, 0)),
            pl.BlockSpec((_BN, wp), lambda i: (i, 0)),
            pl.BlockSpec((1, 1), lambda i: (0, 0)),
            pl.BlockSpec((d1p, d1p), lambda i: (0, 0)),
            pl.BlockSpec((1, d1p), lambda i: (0, 0)),
            pl.BlockSpec((d1p, 64), lambda i: (0, 0)),
            pl.BlockSpec((1, 64), lambda i: (0, 0)),
        ],
        out_specs=[pl.BlockSpec((_BN, 32), lambda i: (i, 0)),
                   pl.BlockSpec((_BN, 32), lambda i: (i, 0)),
                   pl.BlockSpec((8, 64), lambda i: (0, 0))],
        out_shape=[jax.ShapeDtypeStruct((N, 32), jnp.float32),
                   jax.ShapeDtypeStruct((N, 32), jnp.float32),
                   jax.ShapeDtypeStruct((8, 64), jnp.float32)],
    )(hlo, hhi, alo, ahi, eps, m1w, m1b, m2w, m2b)


def _nk2_body(ulo, uhi, stats, gamma, beta, olo, ohi):
    u = jnp.concatenate([ulo[...], uhi[...]], axis=1)
    mu = stats[0:1, :] * (1.0 / N)
    var = stats[1:2, :] * (1.0 / N) - mu * mu
    hn = (u - mu) * lax.rsqrt(var + 1e-5) * gamma[...] + beta[...]
    hn = jnp.maximum(hn, 0.0)
    olo[...] = hn[:, :32]
    ohi[...] = hn[:, 32:]


def _bn_relu(ulo, uhi, stats, gamma, beta):
    grid = (N // _BN,)
    return pl.pallas_call(
        _nk2_body,
        grid=grid,
        in_specs=[
            pl.BlockSpec((_BN, 32), lambda i: (i, 0)),
            pl.BlockSpec((_BN, 32), lambda i: (i, 0)),
            pl.BlockSpec((8, 64), lambda i: (0, 0)),
            pl.BlockSpec((1, 64), lambda i: (0, 0)),
            pl.BlockSpec((1, 64), lambda i: (0, 0)),
        ],
        out_specs=[pl.BlockSpec((_BN, 32), lambda i: (i, 0)),
                   pl.BlockSpec((_BN, 32), lambda i: (i, 0))],
        out_shape=[jax.ShapeDtypeStruct((N, 32), jnp.float32),
                   jax.ShapeDtypeStruct((N, 32), jnp.float32)],
    )(ulo, uhi, stats, gamma, beta)


def _pool_body(hlo, hhi, b_ref, ps, cnt):
    i = pl.program_id(0)
    h = jnp.concatenate([hlo[...], hhi[...]], axis=1)
    b = b_ref[...]
    ids = lax.broadcasted_iota(jnp.int32, (_BN, G), 1)
    oh = jnp.where(b == ids, 1.0, 0.0).astype(jnp.float32)

    @pl.when(i == 0)
    def _():
        ps[...] = jnp.zeros_like(ps)
        cnt[...] = jnp.zeros_like(cnt)
    ps[...] += _dotT(oh, h)
    cnt[...] += _dotT(oh, jnp.ones((_BN, 1), jnp.float32))


def _pool(hlo, hhi, batch2d):
    grid = (N // _BN,)
    return pl.pallas_call(
        _pool_body,
        grid=grid,
        in_specs=[
            pl.BlockSpec((_BN, 32), lambda i: (i, 0)),
            pl.BlockSpec((_BN, 32), lambda i: (i, 0)),
            pl.BlockSpec((_BN, 1), lambda i: (i, 0)),
        ],
        out_specs=[pl.BlockSpec((G, 64), lambda i: (0, 0)),
                   pl.BlockSpec((G, 1), lambda i: (0, 0))],
        out_shape=[jax.ShapeDtypeStruct((G, 64), jnp.float32),
                   jax.ShapeDtypeStruct((G, 1), jnp.float32)],
    )(hlo, hhi, batch2d)


def _head_body(ps, cnt, w1, b1, w2, b2, w3, b3, w4, b4, out_ref):
    g = ps[...] / jnp.maximum(cnt[...], 1.0)
    g = jnp.maximum(_dot(g, w1[...]) + b1[...], 0.0)
    g = jnp.maximum(_dot(g, w2[...]) + b2[...], 0.0)
    g = jnp.maximum(_dot(g, w3[...]) + b3[...], 0.0)
    out_ref[...] = _dot(g, w4[...]) + b4[...]


def _head(ps, cnt, p):
    return pl.pallas_call(
        _head_body,
        out_shape=jax.ShapeDtypeStruct((G, 1), jnp.float32),
    )(ps, cnt,
      p["fc1"]["w"], p["fc1"]["b"].reshape(1, -1),
      p["fc2"]["w"], p["fc2"]["b"].reshape(1, -1),
      p["fc3"]["w"], p["fc3"]["b"].reshape(1, -1),
      p["fc4"]["w"], p["fc4"]["b"].reshape(1, -1))


def _pad_to(a, rows, cols):
    return jnp.pad(a, ((0, rows - a.shape[0]), (0, cols - a.shape[1])))


def kernel(x, edge_attr, edge_weight, params, edge_index, batch):
    hlo = jnp.pad(x, ((0, 0), (0, 4)))
    hhi = jnp.zeros((N, 32), jnp.float32)

    def stk(fn):
        return jnp.stack([fn(conv_p, bn_p, 28 if li == 0 else 64)
                          for li, (conv_p, bn_p)
                          in enumerate(zip(params["convs"], params["bns"]))])

    lp = {
        "w1": stk(lambda c, b, d: _pad_to(c["be1"]["w"], 3, 64)),
        "b1": stk(lambda c, b, d: jnp.pad(c["be1"]["b"], (0, 64 - d)).reshape(1, -1)),
        "w2": stk(lambda c, b, d: _pad_to(c["be2"]["w"], 64, 64)),
        "b2": stk(lambda c, b, d: jnp.pad(c["be2"]["b"], (0, 64 - d)).reshape(1, -1)),
        "m1w": stk(lambda c, b, d: _pad_to(c["mlp1"]["w"], 64, 64)),
        "m1b": stk(lambda c, b, d: jnp.pad(c["mlp1"]["b"], (0, 64 - d)).reshape(1, -1)),
        "m2w": stk(lambda c, b, d: _pad_to(c["mlp2"]["w"], 64, 64)),
        "m2b": stk(lambda c, b, d: c["mlp2"]["b"].reshape(1, -1)),
        "eps": stk(lambda c, b, d: c["eps"].reshape(1, 1)),
        "gamma": stk(lambda c, b, d: b["gamma"].reshape(1, -1)),
        "beta": stk(lambda c, b, d: b["beta"].reshape(1, -1)),
    }

    msgpass = _make_msgpass(32)

    def body(carry, p):
        hlo, hhi = carry
        elo, ehi = _edge_mlp(edge_attr, p["w1"], p["b1"], p["w2"], p["b2"], 32)
        agg_lo, agg_hi = msgpass(hlo, hhi, elo, ehi, edge_index, edge_weight)
        ulo, uhi, stats = _node_mlp(hlo, hhi, agg_lo, agg_hi, p["eps"],
                                    p["m1w"], p["m1b"], p["m2w"], p["m2b"])
        hlo, hhi = _bn_relu(ulo, uhi, stats, p["gamma"], p["beta"])
        return (hlo, hhi), None

    (hlo, hhi), _ = lax.scan(body, (hlo, hhi), lp)

    ps, cnt = _pool(hlo, hhi, batch.reshape(-1, 1))
    out = _head(ps, cnt, params)
    return out.reshape(-1)


# speedup vs baseline: 1.6839x; 1.4655x over previous
import functools

import jax
import jax.numpy as jnp
from jax import lax
from jax.experimental import pallas as pl
from jax.experimental.pallas import tpu as pltpu
from jax.experimental.pallas import tpu_sc as plsc

N = 50000
E = 800000
G = 256

_HI = jax.lax.Precision.HIGHEST


def _dot(a, b):
    return jax.lax.dot_general(a, b, (((1,), (0,)), ((), ())),
                               precision=_HI, preferred_element_type=jnp.float32)


def _dotT(a, b):
    return jax.lax.dot_general(a, b, (((0,), (0,)), ((), ())),
                               precision=_HI, preferred_element_type=jnp.float32)


@functools.lru_cache(maxsize=None)
def _make_msgpass(w):
    NT = 16
    BLK = 400
    EPT = E // NT
    NBLK = EPT // BLK
    NPT = 3136
    NP = NT * NPT
    ZR = 392
    NH = w // 16

    mesh = plsc.VectorSubcoreMesh(core_axis_name="c", subcore_axis_name="s")

    @functools.partial(
        pl.kernel, mesh=mesh,
        compiler_params=pltpu.CompilerParams(use_tc_tiling_on_sc=False),
        out_type=[jax.ShapeDtypeStruct((NP, w), jnp.float32),
                  jax.ShapeDtypeStruct((NP, w), jnp.float32)],
        scratch_types=[
            pltpu.VMEM((BLK,), jnp.int32),
            pltpu.VMEM((BLK,), jnp.int32),
            pltpu.VMEM((BLK,), jnp.float32),
            pltpu.VMEM((BLK, w), jnp.float32),
            pltpu.VMEM((BLK, w), jnp.float32),
            pltpu.VMEM_SHARED((NP, w), jnp.float32),
            pltpu.SemaphoreType.DMA,
        ],
    )
    def msgpass(hlo, hhi, elo, ehi, ei, ew, out_lo, out_hi,
                src_v, dst_v, w_v, xs_v, e_v, agg, sem):
        c = lax.axis_index("c")
        s = lax.axis_index("s")

        def zrow(i, carry):
            for hh in range(NH):
                xs_v[i, pl.ds(hh * 16, 16)] = jnp.zeros((16,), jnp.float32)
            return carry
        lax.fori_loop(0, ZR, zrow, 0)
        for r in range(NPT // ZR):
            pltpu.sync_copy(xs_v.at[pl.ds(0, ZR)],
                            agg.at[pl.ds(s * NPT + r * ZR, ZR)])
        plsc.subcore_barrier()

        base0 = s * EPT

        def blk(b, carry):
            base = base0 + b * BLK
            pltpu.sync_copy(ei.at[0, pl.ds(base, BLK)], src_v)
            pltpu.sync_copy(ei.at[1, pl.ds(base, BLK)], dst_v)
            pltpu.sync_copy(ew.at[pl.ds(base, BLK)], w_v)

            @pl.when(c == 0)
            def _():
                pltpu.sync_copy(elo.at[pl.ds(base, BLK)], e_v)
                pltpu.async_copy(hlo.at[src_v], xs_v, sem).wait()

            @pl.when(c == 1)
            def _():
                pltpu.sync_copy(ehi.at[pl.ds(base, BLK)], e_v)
                pltpu.async_copy(hhi.at[src_v], xs_v, sem).wait()

            def grp(g, carry2):
                j0 = g * 16
                wv = w_v[pl.ds(j0, 16)]
                for k in range(16):
                    wj = wv[k]
                    for hh in range(NH):
                        t = (xs_v[j0 + k, pl.ds(hh * 16, 16)]
                             + e_v[j0 + k, pl.ds(hh * 16, 16)])
                        xs_v[j0 + k, pl.ds(hh * 16, 16)] = (
                            jnp.maximum(t, 0.0) * wj)
                return carry2
            lax.fori_loop(0, BLK // 16, grp, 0)

            pltpu.sync_copy(xs_v, agg.at[dst_v], add=True)
            return carry
        lax.fori_loop(0, NBLK, blk, 0)

        plsc.subcore_barrier()

        @pl.when(c == 0)
        def _():
            pltpu.sync_copy(agg.at[pl.ds(s * NPT, NPT)],
                            out_lo.at[pl.ds(s * NPT, NPT)])

        @pl.when(c == 1)
        def _():
            pltpu.sync_copy(agg.at[pl.ds(s * NPT, NPT)],
                            out_hi.at[pl.ds(s * NPT, NPT)])

    return msgpass


_BE = 3200
_BN = 2000


def _prep_body(x_ref, olo, ohi):
    xb = x_ref[...]
    olo[...] = jnp.pad(xb, ((0, 0), (0, 4)))
    ohi[...] = jnp.zeros_like(ohi)


def _prep(x):
    return pl.pallas_call(
        _prep_body,
        grid=(N // _BN,),
        in_specs=[pl.BlockSpec((_BN, 28), lambda i: (i, 0))],
        out_specs=[pl.BlockSpec((_BN, 32), lambda i: (i, 0)),
                   pl.BlockSpec((_BN, 32), lambda i: (i, 0))],
        out_shape=[jax.ShapeDtypeStruct((N, 32), jnp.float32),
                   jax.ShapeDtypeStruct((N, 32), jnp.float32)],
    )(x)


def _edge_body(d1, ea_ref, w1, b1, w2, b2, olo, ohi):
    a = ea_ref[...]
    t = jnp.maximum(_dotT(a, w1[...]) + b1[...], 0.0)
    ee = _dot(t, w2[...]) + b2[...]
    if d1 == 28:
        olo[...] = jnp.pad(ee, ((0, 0), (0, 4)))
        ohi[...] = jnp.zeros_like(ohi)
    else:
        olo[...] = ee[:, :32]
        ohi[...] = ee[:, 32:]


def _edge_mlp(ea_t, w1, b1, w2, b2):
    d1 = w1.shape[1]
    return pl.pallas_call(
        functools.partial(_edge_body, d1),
        grid=(E // _BE,),
        in_specs=[
            pl.BlockSpec((3, _BE), lambda i: (0, i)),
            pl.BlockSpec((3, d1), lambda i: (0, 0)),
            pl.BlockSpec((d1,), lambda i: (0,)),
            pl.BlockSpec((d1, d1), lambda i: (0, 0)),
            pl.BlockSpec((d1,), lambda i: (0,)),
        ],
        out_specs=[pl.BlockSpec((_BE, 32), lambda i: (i, 0)),
                   pl.BlockSpec((_BE, 32), lambda i: (i, 0))],
        out_shape=[jax.ShapeDtypeStruct((E, 32), jnp.float32),
                   jax.ShapeDtypeStruct((E, 32), jnp.float32)],
    )(ea_t, w1, b1, w2, b2)


def _nk1_body(d1, hlo, hhi, alo, ahi, eps, m1w, m1b, m2w, m2b,
              ulo, uhi, stats):
    i = pl.program_id(0)
    h = jnp.concatenate([hlo[...], hhi[...]], axis=1)
    agg = jnp.concatenate([alo[...], ahi[...]], axis=1)
    z = (1.0 + eps[0]) * h + agg
    zc = z[:, :d1]
    t = jnp.maximum(_dot(zc, m1w[...]) + m1b[...], 0.0)
    u = _dot(t, m2w[...]) + m2b[...]
    ulo[...] = u[:, :32]
    uhi[...] = u[:, 32:]

    @pl.when(i == 0)
    def _():
        stats[...] = jnp.zeros_like(stats)
    stats[0:1, :] += jnp.sum(u, axis=0, keepdims=True)
    stats[1:2, :] += jnp.sum(u * u, axis=0, keepdims=True)


def _node_mlp(hlo, hhi, alo, ahi, eps, m1w, m1b, m2w, m2b):
    d1 = m1w.shape[0]
    return pl.pallas_call(
        functools.partial(_nk1_body, d1),
        grid=(N // _BN,),
        in_specs=[
            pl.BlockSpec((_BN, 32), lambda i: (i, 0)),
            pl.BlockSpec((_BN, 32), lambda i: (i, 0)),
            pl.BlockSpec((_BN, 32), lambda i: (i, 0)),
            pl.BlockSpec((_BN, 32), lambda i: (i, 0)),
            pl.BlockSpec(memory_space=pltpu.SMEM),
            pl.BlockSpec((d1, d1), lambda i: (0, 0)),
            pl.BlockSpec((d1,), lambda i: (0,)),
            pl.BlockSpec((d1, 64), lambda i: (0, 0)),
            pl.BlockSpec((64,), lambda i: (0,)),
        ],
        out_specs=[pl.BlockSpec((_BN, 32), lambda i: (i, 0)),
                   pl.BlockSpec((_BN, 32), lambda i: (i, 0)),
                   pl.BlockSpec((8, 64), lambda i: (0, 0))],
        out_shape=[jax.ShapeDtypeStruct((N, 32), jnp.float32),
                   jax.ShapeDtypeStruct((N, 32), jnp.float32),
                   jax.ShapeDtypeStruct((8, 64), jnp.float32)],
    )(hlo, hhi, alo, ahi, eps, m1w, m1b, m2w, m2b)


def _nk2_body(ulo, uhi, stats, gamma, beta, olo, ohi):
    u = jnp.concatenate([ulo[...], uhi[...]], axis=1)
    mu = stats[0:1, :] * (1.0 / N)
    var = stats[1:2, :] * (1.0 / N) - mu * mu
    hn = (u - mu) * lax.rsqrt(var + 1e-5) * gamma[...] + beta[...]
    hn = jnp.maximum(hn, 0.0)
    olo[...] = hn[:, :32]
    ohi[...] = hn[:, 32:]


def _bn_relu(ulo, uhi, stats, gamma, beta):
    return pl.pallas_call(
        _nk2_body,
        grid=(N // _BN,),
        in_specs=[
            pl.BlockSpec((_BN, 32), lambda i: (i, 0)),
            pl.BlockSpec((_BN, 32), lambda i: (i, 0)),
            pl.BlockSpec((8, 64), lambda i: (0, 0)),
            pl.BlockSpec((64,), lambda i: (0,)),
            pl.BlockSpec((64,), lambda i: (0,)),
        ],
        out_specs=[pl.BlockSpec((_BN, 32), lambda i: (i, 0)),
                   pl.BlockSpec((_BN, 32), lambda i: (i, 0))],
        out_shape=[jax.ShapeDtypeStruct((N, 32), jnp.float32),
                   jax.ShapeDtypeStruct((N, 32), jnp.float32)],
    )(ulo, uhi, stats, gamma, beta)


def _pool_body(hlo, hhi, b_ref, ps, cnt):
    i = pl.program_id(0)
    h = jnp.concatenate([hlo[...], hhi[...]], axis=1)
    b = b_ref[...]
    ids = lax.broadcasted_iota(jnp.int32, (_BN, G), 1)
    oh = jnp.where(b == ids, 1.0, 0.0).astype(jnp.float32)

    @pl.when(i == 0)
    def _():
        ps[...] = jnp.zeros_like(ps)
        cnt[...] = jnp.zeros_like(cnt)
    ps[...] += _dotT(oh, h)
    cnt[...] += _dotT(oh, jnp.ones((_BN, 1), jnp.float32))


def _pool(hlo, hhi, batch):
    return pl.pallas_call(
        _pool_body,
        grid=(N // _BN,),
        in_specs=[
            pl.BlockSpec((_BN, 32), lambda i: (i, 0)),
            pl.BlockSpec((_BN, 32), lambda i: (i, 0)),
            pl.BlockSpec((_BN, 1), lambda i: (i, 0)),
        ],
        out_specs=[pl.BlockSpec((G, 64), lambda i: (0, 0)),
                   pl.BlockSpec((G, 1), lambda i: (0, 0))],
        out_shape=[jax.ShapeDtypeStruct((G, 64), jnp.float32),
                   jax.ShapeDtypeStruct((G, 1), jnp.float32)],
    )(hlo, hhi, batch)


def _head_body(ps, cnt, w1, b1, w2, b2, w3, b3, w4, b4, out_ref):
    g = ps[...] / jnp.maximum(cnt[...], 1.0)
    g = jnp.maximum(_dot(g, w1[...]) + b1[...], 0.0)
    g = jnp.maximum(_dot(g, w2[...]) + b2[...], 0.0)
    g = jnp.maximum(_dot(g, w3[...]) + b3[...], 0.0)
    out_ref[...] = _dot(g, w4[...]) + b4[...]


def _head(ps, cnt, p):
    return pl.pallas_call(
        _head_body,
        out_shape=jax.ShapeDtypeStruct((G, 1), jnp.float32),
    )(ps, cnt,
      p["fc1"]["w"], p["fc1"]["b"], p["fc2"]["w"], p["fc2"]["b"],
      p["fc3"]["w"], p["fc3"]["b"], p["fc4"]["w"], p["fc4"]["b"])


def kernel(x, edge_attr, edge_weight, params, edge_index, batch):
    hlo, hhi = _prep(x)
    ea_t = edge_attr.T
    msgpass = _make_msgpass(32)

    for conv_p, bn_p in zip(params["convs"], params["bns"]):
        elo, ehi = _edge_mlp(ea_t, conv_p["be1"]["w"], conv_p["be1"]["b"],
                             conv_p["be2"]["w"], conv_p["be2"]["b"])
        agg_lo, agg_hi = msgpass(hlo, hhi, elo, ehi, edge_index, edge_weight)
        ulo, uhi, stats = _node_mlp(hlo, hhi, agg_lo, agg_hi, conv_p["eps"],
                                    conv_p["mlp1"]["w"], conv_p["mlp1"]["b"],
                                    conv_p["mlp2"]["w"], conv_p["mlp2"]["b"])
        hlo, hhi = _bn_relu(ulo, uhi, stats, bn_p["gamma"], bn_p["beta"])

    ps, cnt = _pool(hlo, hhi, batch.reshape(-1, 1))
    out = _head(ps, cnt, params)
    return out.reshape(-1)


# speedup vs baseline: 3.2186x; 1.9114x over previous
import functools

import jax
import jax.numpy as jnp
from jax import lax
from jax.experimental import pallas as pl
from jax.experimental.pallas import tpu as pltpu
from jax.experimental.pallas import tpu_sc as plsc

N = 50000
E = 800000
G = 256

_HI = jax.lax.Precision.HIGHEST
_LO = jax.lax.Precision.DEFAULT


def _dot(a, b, prec=_HI):
    return jax.lax.dot_general(a, b, (((1,), (0,)), ((), ())),
                               precision=prec, preferred_element_type=jnp.float32)


def _dotT(a, b, prec=_HI):
    return jax.lax.dot_general(a, b, (((0,), (0,)), ((), ())),
                               precision=prec, preferred_element_type=jnp.float32)


@functools.lru_cache(maxsize=None)
def _make_msgpass(w):
    NT = 16
    BLK = 400
    EPT = E // NT
    NBLK = EPT // BLK
    NPT = 3136
    NP = NT * NPT
    ZR = 392
    NH = w // 16

    mesh = plsc.VectorSubcoreMesh(core_axis_name="c", subcore_axis_name="s")

    @functools.partial(
        pl.kernel, mesh=mesh,
        compiler_params=pltpu.CompilerParams(use_tc_tiling_on_sc=False),
        out_type=[jax.ShapeDtypeStruct((NP, w), jnp.float32),
                  jax.ShapeDtypeStruct((NP, w), jnp.float32)],
        scratch_types=[
            pltpu.VMEM((BLK,), jnp.int32),
            pltpu.VMEM((BLK,), jnp.int32),
            pltpu.VMEM((BLK,), jnp.float32),
            pltpu.VMEM((BLK, w), jnp.float32),
            pltpu.VMEM((BLK, w), jnp.float32),
            pltpu.VMEM_SHARED((NP, w), jnp.float32),
            pltpu.SemaphoreType.DMA,
            pltpu.SemaphoreType.DMA,
            pltpu.SemaphoreType.DMA,
        ],
    )
    def msgpass(hlo, hhi, elo, ehi, ei, ew, out_lo, out_hi,
                src_v, dst_v, w_v, xs_v, e_v, agg, sem, sem2, sem3):
        c = lax.axis_index("c")
        s = lax.axis_index("s")

        def zrow(i, carry):
            for hh in range(NH):
                xs_v[i, pl.ds(hh * 16, 16)] = jnp.zeros((16,), jnp.float32)
            return carry
        lax.fori_loop(0, ZR, zrow, 0)
        for r in range(NPT // ZR):
            pltpu.sync_copy(xs_v.at[pl.ds(0, ZR)],
                            agg.at[pl.ds(s * NPT + r * ZR, ZR)])
        plsc.subcore_barrier()

        base0 = s * EPT

        def blk(b, carry):
            base = base0 + b * BLK
            pltpu.async_copy(ei.at[0, pl.ds(base, BLK)], src_v, sem)
            pltpu.async_copy(ei.at[1, pl.ds(base, BLK)], dst_v, sem)
            pltpu.async_copy(ew.at[pl.ds(base, BLK)], w_v, sem)

            @pl.when(c == 0)
            def _():
                pltpu.async_copy(elo.at[pl.ds(base, BLK)], e_v, sem)

            @pl.when(c == 1)
            def _():
                pltpu.async_copy(ehi.at[pl.ds(base, BLK)], e_v, sem)

            pltpu.make_async_copy(ei.at[0, pl.ds(base, BLK)], src_v, sem).wait()
            pltpu.make_async_copy(ei.at[1, pl.ds(base, BLK)], dst_v, sem).wait()
            pltpu.make_async_copy(ew.at[pl.ds(base, BLK)], w_v, sem).wait()
            pltpu.make_async_copy(elo.at[pl.ds(base, BLK)], e_v, sem).wait()

            HB = BLK // 2

            @pl.when(c == 0)
            def _():
                pltpu.async_copy(hlo.at[src_v.at[pl.ds(0, HB)]],
                                 xs_v.at[pl.ds(0, HB)], sem2)
                pltpu.async_copy(hlo.at[src_v.at[pl.ds(HB, HB)]],
                                 xs_v.at[pl.ds(HB, HB)], sem3)

            @pl.when(c == 1)
            def _():
                pltpu.async_copy(hhi.at[src_v.at[pl.ds(0, HB)]],
                                 xs_v.at[pl.ds(0, HB)], sem2)
                pltpu.async_copy(hhi.at[src_v.at[pl.ds(HB, HB)]],
                                 xs_v.at[pl.ds(HB, HB)], sem3)

            def grp(g, carry2):
                j0 = g * 16
                wv = w_v[pl.ds(j0, 16)]
                for k in range(16):
                    wj = wv[k]
                    for hh in range(NH):
                        t = (xs_v[j0 + k, pl.ds(hh * 16, 16)]
                             + e_v[j0 + k, pl.ds(hh * 16, 16)])
                        xs_v[j0 + k, pl.ds(hh * 16, 16)] = (
                            jnp.maximum(t, 0.0) * wj)
                return carry2

            @pl.when(c == 0)
            def _():
                pltpu.make_async_copy(hlo.at[src_v.at[pl.ds(0, HB)]],
                                      xs_v.at[pl.ds(0, HB)], sem2).wait()

            @pl.when(c == 1)
            def _():
                pltpu.make_async_copy(hhi.at[src_v.at[pl.ds(0, HB)]],
                                      xs_v.at[pl.ds(0, HB)], sem2).wait()

            lax.fori_loop(0, BLK // 32, grp, 0)

            @pl.when(c == 0)
            def _():
                pltpu.make_async_copy(hlo.at[src_v.at[pl.ds(HB, HB)]],
                                      xs_v.at[pl.ds(HB, HB)], sem3).wait()

            @pl.when(c == 1)
            def _():
                pltpu.make_async_copy(hhi.at[src_v.at[pl.ds(HB, HB)]],
                                      xs_v.at[pl.ds(HB, HB)], sem3).wait()

            lax.fori_loop(BLK // 32, BLK // 16, grp, 0)

            pltpu.sync_copy(xs_v, agg.at[dst_v], add=True)
            return carry
        lax.fori_loop(0, NBLK, blk, 0)

        plsc.subcore_barrier()

        @pl.when(c == 0)
        def _():
            pltpu.sync_copy(agg.at[pl.ds(s * NPT, NPT)],
                            out_lo.at[pl.ds(s * NPT, NPT)])

        @pl.when(c == 1)
        def _():
            pltpu.sync_copy(agg.at[pl.ds(s * NPT, NPT)],
                            out_hi.at[pl.ds(s * NPT, NPT)])

    return msgpass


_BE = 3200
_BN = 2000


def _prep_body(x_ref, olo, ohi):
    xb = x_ref[...]
    olo[...] = jnp.pad(xb, ((0, 0), (0, 4)))
    ohi[...] = jnp.zeros_like(ohi)


def _prep(x):
    return pl.pallas_call(
        _prep_body,
        grid=(N // _BN,),
        in_specs=[pl.BlockSpec((_BN, 28), lambda i: (i, 0))],
        out_specs=[pl.BlockSpec((_BN, 32), lambda i: (i, 0)),
                   pl.BlockSpec((_BN, 32), lambda i: (i, 0))],
        out_shape=[jax.ShapeDtypeStruct((N, 32), jnp.float32),
                   jax.ShapeDtypeStruct((N, 32), jnp.float32)],
    )(x)


def _edge4_body(ea_ref, *refs):
    ws = refs[:16]
    outs = refs[16:]
    a = ea_ref[...]

    def padw(m, rows, cols):
        v = m[...]
        return jnp.pad(v, ((0, rows - v.shape[0]), (0, cols - v.shape[1])))

    def padb(bref):
        v = bref[...]
        return jnp.pad(v, (0, 64 - v.shape[0]))

    w1all = jnp.concatenate([padw(ws[4 * l], 3, 64) for l in range(4)], axis=1)
    b1all = jnp.concatenate([padb(ws[4 * l + 1]) for l in range(4)])
    bands = []
    for l in range(4):
        w2p = padw(ws[4 * l + 2], 64, 64)
        pieces = []
        if l > 0:
            pieces.append(jnp.zeros((64, 64 * l), jnp.float32))
        pieces.append(w2p)
        if l < 3:
            pieces.append(jnp.zeros((64, 64 * (3 - l)), jnp.float32))
        bands.append(jnp.concatenate(pieces, axis=1))
    w2all = jnp.concatenate(bands, axis=0)
    b2all = jnp.concatenate([padb(ws[4 * l + 3]) for l in range(4)])

    t = jnp.maximum(_dotT(a, w1all, _LO) + b1all, 0.0)
    ee = _dot(t, w2all, _LO) + b2all
    for l in range(4):
        outs[2 * l][...] = ee[:, 64 * l:64 * l + 32]
        outs[2 * l + 1][...] = ee[:, 64 * l + 32:64 * (l + 1)]


def _edge_mlp4(ea_t, convs):
    args = [ea_t]
    in_specs = [pl.BlockSpec((3, _BE), lambda i: (0, i))]
    for conv_p in convs:
        d1 = conv_p["be1"]["w"].shape[1]
        args += [conv_p["be1"]["w"], conv_p["be1"]["b"],
                 conv_p["be2"]["w"], conv_p["be2"]["b"]]
        in_specs += [pl.BlockSpec((3, d1), lambda i: (0, 0)),
                     pl.BlockSpec((d1,), lambda i: (0,)),
                     pl.BlockSpec((d1, d1), lambda i: (0, 0)),
                     pl.BlockSpec((d1,), lambda i: (0,))]
    return pl.pallas_call(
        _edge4_body,
        grid=(E // _BE,),
        in_specs=in_specs,
        out_specs=[pl.BlockSpec((_BE, 32), lambda i: (i, 0))] * 8,
        out_shape=[jax.ShapeDtypeStruct((E, 32), jnp.float32)] * 8,
    )(*args)


def _nk1_body(d1, hlo, hhi, alo, ahi, eps, m1w, m1b, m2w, m2b,
              ulo, uhi, stats):
    i = pl.program_id(0)
    h = jnp.concatenate([hlo[...], hhi[...]], axis=1)
    agg = jnp.concatenate([alo[...], ahi[...]], axis=1)
    z = (1.0 + eps[0]) * h + agg
    zc = z[:, :d1]
    t = jnp.maximum(_dot(zc, m1w[...]) + m1b[...], 0.0)
    u = _dot(t, m2w[...]) + m2b[...]
    ulo[...] = u[:, :32]
    uhi[...] = u[:, 32:]

    @pl.when(i == 0)
    def _():
        stats[...] = jnp.zeros_like(stats)
    stats[0:1, :] += jnp.sum(u, axis=0, keepdims=True)
    stats[1:2, :] += jnp.sum(u * u, axis=0, keepdims=True)


def _node_mlp(hlo, hhi, alo, ahi, eps, m1w, m1b, m2w, m2b):
    d1 = m1w.shape[0]
    return pl.pallas_call(
        functools.partial(_nk1_body, d1),
        grid=(N // _BN,),
        in_specs=[
            pl.BlockSpec((_BN, 32), lambda i: (i, 0)),
            pl.BlockSpec((_BN, 32), lambda i: (i, 0)),
            pl.BlockSpec((_BN, 32), lambda i: (i, 0)),
            pl.BlockSpec((_BN, 32), lambda i: (i, 0)),
            pl.BlockSpec(memory_space=pltpu.SMEM),
            pl.BlockSpec((d1, d1), lambda i: (0, 0)),
            pl.BlockSpec((d1,), lambda i: (0,)),
            pl.BlockSpec((d1, 64), lambda i: (0, 0)),
            pl.BlockSpec((64,), lambda i: (0,)),
        ],
        out_specs=[pl.BlockSpec((_BN, 32), lambda i: (i, 0)),
                   pl.BlockSpec((_BN, 32), lambda i: (i, 0)),
                   pl.BlockSpec((8, 64), lambda i: (0, 0))],
        out_shape=[jax.ShapeDtypeStruct((N, 32), jnp.float32),
                   jax.ShapeDtypeStruct((N, 32), jnp.float32),
                   jax.ShapeDtypeStruct((8, 64), jnp.float32)],
    )(hlo, hhi, alo, ahi, eps, m1w, m1b, m2w, m2b)


def _nk2_body(ulo, uhi, stats, gamma, beta, olo, ohi):
    u = jnp.concatenate([ulo[...], uhi[...]], axis=1)
    mu = stats[0:1, :] * (1.0 / N)
    var = stats[1:2, :] * (1.0 / N) - mu * mu
    hn = (u - mu) * lax.rsqrt(var + 1e-5) * gamma[...] + beta[...]
    hn = jnp.maximum(hn, 0.0)
    olo[...] = hn[:, :32]
    ohi[...] = hn[:, 32:]


def _bn_relu(ulo, uhi, stats, gamma, beta):
    return pl.pallas_call(
        _nk2_body,
        grid=(N // _BN,),
        in_specs=[
            pl.BlockSpec((_BN, 32), lambda i: (i, 0)),
            pl.BlockSpec((_BN, 32), lambda i: (i, 0)),
            pl.BlockSpec((8, 64), lambda i: (0, 0)),
            pl.BlockSpec((64,), lambda i: (0,)),
            pl.BlockSpec((64,), lambda i: (0,)),
        ],
        out_specs=[pl.BlockSpec((_BN, 32), lambda i: (i, 0)),
                   pl.BlockSpec((_BN, 32), lambda i: (i, 0))],
        out_shape=[jax.ShapeDtypeStruct((N, 32), jnp.float32),
                   jax.ShapeDtypeStruct((N, 32), jnp.float32)],
    )(ulo, uhi, stats, gamma, beta)


def _pool_body(hlo, hhi, b_ref, ps, cnt):
    i = pl.program_id(0)
    h = jnp.concatenate([hlo[...], hhi[...]], axis=1)
    b = b_ref[...]
    ids = lax.broadcasted_iota(jnp.int32, (_BN, G), 1)
    oh = jnp.where(b == ids, 1.0, 0.0).astype(jnp.float32)

    @pl.when(i == 0)
    def _():
        ps[...] = jnp.zeros_like(ps)
        cnt[...] = jnp.zeros_like(cnt)
    ps[...] += _dotT(oh, h, _LO)
    cnt[...] += _dotT(oh, jnp.ones((_BN, 1), jnp.float32), _LO)


def _pool(hlo, hhi, batch):
    return pl.pallas_call(
        _pool_body,
        grid=(N // _BN,),
        in_specs=[
            pl.BlockSpec((_BN, 32), lambda i: (i, 0)),
            pl.BlockSpec((_BN, 32), lambda i: (i, 0)),
            pl.BlockSpec((_BN, 1), lambda i: (i, 0)),
        ],
        out_specs=[pl.BlockSpec((G, 64), lambda i: (0, 0)),
                   pl.BlockSpec((G, 1), lambda i: (0, 0))],
        out_shape=[jax.ShapeDtypeStruct((G, 64), jnp.float32),
                   jax.ShapeDtypeStruct((G, 1), jnp.float32)],
    )(hlo, hhi, batch)


def _head_body(ps, cnt, w1, b1, w2, b2, w3, b3, w4, b4, out_ref):
    g = ps[...] / jnp.maximum(cnt[...], 1.0)
    g = jnp.maximum(_dot(g, w1[...]) + b1[...], 0.0)
    g = jnp.maximum(_dot(g, w2[...]) + b2[...], 0.0)
    g = jnp.maximum(_dot(g, w3[...]) + b3[...], 0.0)
    out_ref[...] = _dot(g, w4[...]) + b4[...]


def _head(ps, cnt, p):
    return pl.pallas_call(
        _head_body,
        out_shape=jax.ShapeDtypeStruct((G, 1), jnp.float32),
    )(ps, cnt,
      p["fc1"]["w"], p["fc1"]["b"], p["fc2"]["w"], p["fc2"]["b"],
      p["fc3"]["w"], p["fc3"]["b"], p["fc4"]["w"], p["fc4"]["b"])


def kernel(x, edge_attr, edge_weight, params, edge_index, batch):
    hlo, hhi = _prep(x)
    ea_t = edge_attr.T
    msgpass = _make_msgpass(32)
    e_all = _edge_mlp4(ea_t, params["convs"])

    for li, (conv_p, bn_p) in enumerate(zip(params["convs"], params["bns"])):
        elo, ehi = e_all[2 * li], e_all[2 * li + 1]
        agg_lo, agg_hi = msgpass(hlo, hhi, elo, ehi, edge_index, edge_weight)
        ulo, uhi, stats = _node_mlp(hlo, hhi, agg_lo, agg_hi, conv_p["eps"],
                                    conv_p["mlp1"]["w"], conv_p["mlp1"]["b"],
                                    conv_p["mlp2"]["w"], conv_p["mlp2"]["b"])
        hlo, hhi = _bn_relu(ulo, uhi, stats, bn_p["gamma"], bn_p["beta"])

    ps, cnt = _pool(hlo, hhi, batch.reshape(-1, 1))
    out = _head(ps, cnt, params)
    return out.reshape(-1)
